# baseline (speedup 1.0000x reference)
import functools

import jax
import jax.numpy as jnp
from jax import lax
from jax.experimental import pallas as pl
from jax.experimental.pallas import tpu as pltpu
from jax.experimental.pallas import tpu_sc as plsc

DIM = 64
NC = 2
NS = 16
NW = NC * NS


@functools.partial(jax.jit, static_argnames=("b_per_w", "chunk", "nchunk"))
def _gather(table, idx, b_per_w, chunk, nchunk):
    B = idx.shape[0]
    mesh = plsc.VectorSubcoreMesh(core_axis_name="c", subcore_axis_name="s")

    @functools.partial(
        pl.kernel,
        mesh=mesh,
        out_type=jax.ShapeDtypeStruct((B, DIM), jnp.float32),
        compiler_params=pltpu.CompilerParams(use_tc_tiling_on_sc=False),
        scratch_types=[
            pltpu.VMEM((chunk,), jnp.int32),
            pltpu.VMEM((chunk,), jnp.int32),
            pltpu.VMEM((2, chunk, DIM), jnp.float32),
            pltpu.SemaphoreType.DMA,
        ],
    )
    def gather_k(table_hbm, idx_hbm, out_hbm, idx_v0, idx_v1, rows_v, sem_g):
        wid = lax.axis_index("s") * NC + lax.axis_index("c")
        base = wid * b_per_w

        def body(i, carry):
            for s, idx_v in ((0, idx_v0), (1, idx_v1)):
                off = base + (2 * i + s) * chunk
                pltpu.sync_copy(idx_hbm.at[pl.ds(off, chunk)], idx_v)
                pltpu.async_copy(table_hbm.at[idx_v], rows_v.at[s], sem_g).wait()
                pltpu.sync_copy(rows_v.at[s], out_hbm.at[pl.ds(off, chunk)])
            return carry

        lax.fori_loop(0, nchunk // 2, body, 0)

    return gather_k(table, idx)


def kernel(cat_ids, table):
    batch, fields = cat_ids.shape
    B = batch * fields
    idx = cat_ids.reshape(B).astype(jnp.int32)
    b_per_w = B // NW
    chunk = 512
    nchunk = b_per_w // chunk
    out = _gather(table, idx, b_per_w, chunk, nchunk)
    return out.reshape(batch, fields, DIM)
